# initial kernel scaffold (unmeasured)
import jax
import jax.numpy as jnp
from jax import lax
from jax.experimental import pallas as pl
from jax.experimental.pallas import tpu as pltpu

M = 2048
K = 2048
N = 8192
BLK = 1024
GRID = N // BLK
HALF = M // 2


def kernel(x, dy):
    def body(x_ref, dy_ref, out_ref, partial_ref, recv_ref, send_sems, recv_sems):
        j = pl.program_id(0)
        my_x = lax.axis_index("x")
        my_y = lax.axis_index("y")
        my_z = lax.axis_index("z")
        partner = (my_x, my_y, 1 - my_z)

        @pl.when(j == 0)
        def _():
            barrier = pltpu.get_barrier_semaphore()
            pl.semaphore_signal(
                barrier, inc=1, device_id=partner,
                device_id_type=pltpu.DeviceIdType.MESH,
            )
            pl.semaphore_wait(barrier, 1)

        partial_ref[...] = lax.dot_general(
            x_ref[...], dy_ref[...],
            (((0,), (0,)), ((), ())),
            preferred_element_type=jnp.float32,
        )

        slot = j % 2
        rdma = pltpu.make_async_remote_copy(
            src_ref=partial_ref.at[pl.ds((1 - my_z) * HALF, HALF), :],
            dst_ref=recv_ref.at[slot],
            send_sem=send_sems.at[slot],
            recv_sem=recv_sems.at[slot],
            device_id=partner,
            device_id_type=pltpu.DeviceIdType.MESH,
        )
        rdma.start()
        rdma.wait()

        out_ref[...] = partial_ref[pl.ds(my_z * HALF, HALF), :] + recv_ref[slot]

    return pl.pallas_call(
        body,
        grid=(GRID,),
        in_specs=[
            pl.BlockSpec((K, M), lambda j: (0, 0)),
            pl.BlockSpec((K, BLK), lambda j: (0, j)),
        ],
        out_specs=pl.BlockSpec((HALF, BLK), lambda j: (0, j)),
        out_shape=jax.ShapeDtypeStruct((HALF, N), jnp.float32),
        scratch_shapes=[
            pltpu.VMEM((M, BLK), jnp.float32),
            pltpu.VMEM((2, HALF, BLK), jnp.float32),
            pltpu.SemaphoreType.DMA((2,)),
            pltpu.SemaphoreType.DMA((2,)),
        ],
        compiler_params=pltpu.CompilerParams(
            dimension_semantics=("arbitrary",),
            collective_id=0,
        ),
    )(x, dy)


# baseline (device time: 508147 ns/iter reference)
import jax
import jax.numpy as jnp
from jax import lax
from jax.experimental import pallas as pl
from jax.experimental.pallas import tpu as pltpu

M = 2048
K = 2048
N = 8192
BLK = 512
GRID = N // BLK
HALF = M // 2


def kernel(x, dy):
    def body(x_ref, dy_ref, out_ref, partial_ref, recv_ref, send_sems, recv_sems):
        j = pl.program_id(0)
        my_x = lax.axis_index("x")
        my_y = lax.axis_index("y")
        my_z = lax.axis_index("z")
        partner = (my_x, my_y, 1 - my_z)

        @pl.when(j == 0)
        def _():
            barrier = pltpu.get_barrier_semaphore()
            pl.semaphore_signal(
                barrier, inc=1, device_id=partner,
                device_id_type=pltpu.DeviceIdType.MESH,
            )
            pl.semaphore_wait(barrier, 1)

        partial_ref[...] = lax.dot_general(
            x_ref[...], dy_ref[...],
            (((0,), (0,)), ((), ())),
            preferred_element_type=jnp.float32,
        )

        slot = j % 2
        rdma = pltpu.make_async_remote_copy(
            src_ref=partial_ref.at[pl.ds((1 - my_z) * HALF, HALF), :],
            dst_ref=recv_ref.at[slot],
            send_sem=send_sems.at[slot],
            recv_sem=recv_sems.at[slot],
            device_id=partner,
            device_id_type=pltpu.DeviceIdType.MESH,
        )
        rdma.start()
        rdma.wait()

        out_ref[...] = partial_ref[pl.ds(my_z * HALF, HALF), :] + recv_ref[slot]

    return pl.pallas_call(
        body,
        grid=(GRID,),
        in_specs=[
            pl.BlockSpec((K, M), lambda j: (0, 0)),
            pl.BlockSpec((K, BLK), lambda j: (0, j)),
        ],
        out_specs=pl.BlockSpec((HALF, BLK), lambda j: (0, j)),
        out_shape=jax.ShapeDtypeStruct((HALF, N), jnp.float32),
        scratch_shapes=[
            pltpu.VMEM((M, BLK), jnp.float32),
            pltpu.VMEM((2, HALF, BLK), jnp.float32),
            pltpu.SemaphoreType.DMA((2,)),
            pltpu.SemaphoreType.DMA((2,)),
        ],
        compiler_params=pltpu.CompilerParams(
            dimension_semantics=("arbitrary",),
            collective_id=0,
            vmem_limit_bytes=100 * 1024 * 1024,
        ),
    )(x, dy)


# device time: 260014 ns/iter; 1.9543x vs baseline; 1.9543x over previous
import jax
import jax.numpy as jnp
from jax import lax
from jax.experimental import pallas as pl
from jax.experimental.pallas import tpu as pltpu

M = 2048
K = 2048
N = 8192
HCOLS = N // 2
BLK = 512
NBLK = HCOLS // BLK
HALF = M // 2


def kernel(x, dy):
    def body(x_ref, dy_hbm, out_ref,
             dy_v, partial, zrecv, s_buf,
             dy_sems, z_send_sems, z_recv_sems,
             px_send_sems, py_send_sems, px_recv_sems, py_recv_sems,
             store_sems):
        my_x = lax.axis_index("x")
        my_y = lax.axis_index("y")
        my_z = lax.axis_index("z")
        H = (my_x + my_y) % 2
        col0 = H * HCOLS
        ccol0 = (1 - H) * HCOLS
        z_nbr = (my_x, my_y, 1 - my_z)
        x_nbr = (1 - my_x, my_y, my_z)
        y_nbr = (my_x, 1 - my_y, my_z)

        barrier = pltpu.get_barrier_semaphore()
        for nbr in (z_nbr, x_nbr, y_nbr):
            pl.semaphore_signal(
                barrier, inc=1, device_id=nbr,
                device_id_type=pltpu.DeviceIdType.MESH,
            )
        pl.semaphore_wait(barrier, 3)

        def dy_copy(b):
            return pltpu.make_async_copy(
                dy_hbm.at[:, pl.ds(col0 + b * BLK, BLK)],
                dy_v.at[b % 2],
                dy_sems.at[b % 2],
            )

        def z_rdma(b):
            return pltpu.make_async_remote_copy(
                src_ref=partial.at[b % 2, pl.ds((1 - my_z) * HALF, HALF), :],
                dst_ref=zrecv.at[b % 2],
                send_sem=z_send_sems.at[b % 2],
                recv_sem=z_recv_sems.at[b % 2],
                device_id=z_nbr,
                device_id_type=pltpu.DeviceIdType.MESH,
            )

        def plane_rdma(b):
            nbr = x_nbr if b < NBLK // 2 else y_nbr
            ssems = px_send_sems if b < NBLK // 2 else py_send_sems
            rsems = px_recv_sems if b < NBLK // 2 else py_recv_sems
            return pltpu.make_async_remote_copy(
                src_ref=s_buf.at[b % 2],
                dst_ref=out_ref.at[:, pl.ds(col0 + b * BLK, BLK)],
                send_sem=ssems.at[b % 2],
                recv_sem=rsems.at[b % (NBLK // 2)],
                device_id=nbr,
                device_id_type=pltpu.DeviceIdType.MESH,
            )

        def store_copy(b):
            return pltpu.make_async_copy(
                s_buf.at[b % 2],
                out_ref.at[:, pl.ds(col0 + b * BLK, BLK)],
                store_sems.at[b % 2],
            )

        def finalize(b):
            z_rdma(b).wait_recv()
            if b >= 2:
                plane_rdma(b - 2).wait_send()
                store_copy(b - 2).wait()
            s_buf[b % 2] = (
                partial[b % 2, pl.ds(my_z * HALF, HALF), :] + zrecv[b % 2]
            )
            store_copy(b).start()
            plane_rdma(b).start()

        dy_copy(0).start()
        for b in range(NBLK):
            if b + 1 < NBLK:
                dy_copy(b + 1).start()
            dy_copy(b).wait()
            if b >= 2:
                z_rdma(b - 2).wait_send()
            partial[b % 2] = lax.dot_general(
                x_ref[...], dy_v[b % 2],
                (((0,), (0,)), ((), ())),
                preferred_element_type=jnp.float32,
            )
            if b >= 1:
                finalize(b - 1)
            z_rdma(b).start()
        finalize(NBLK - 1)

        z_rdma(NBLK - 2).wait_send()
        z_rdma(NBLK - 1).wait_send()
        for b in (NBLK - 2, NBLK - 1):
            plane_rdma(b).wait_send()
            store_copy(b).wait()

        for b in range(NBLK):
            nbr = x_nbr if b < NBLK // 2 else y_nbr
            rsems = px_recv_sems if b < NBLK // 2 else py_recv_sems
            recv = pltpu.make_async_remote_copy(
                src_ref=s_buf.at[b % 2],
                dst_ref=out_ref.at[:, pl.ds(ccol0 + b * BLK, BLK)],
                send_sem=px_send_sems.at[b % 2],
                recv_sem=rsems.at[b % (NBLK // 2)],
                device_id=nbr,
                device_id_type=pltpu.DeviceIdType.MESH,
            )
            recv.wait_recv()

    return pl.pallas_call(
        body,
        in_specs=[
            pl.BlockSpec(memory_space=pltpu.MemorySpace.VMEM),
            pl.BlockSpec(memory_space=pl.ANY),
        ],
        out_specs=pl.BlockSpec(memory_space=pl.ANY),
        out_shape=jax.ShapeDtypeStruct((HALF, N), jnp.float32),
        scratch_shapes=[
            pltpu.VMEM((2, K, BLK), jnp.float32),
            pltpu.VMEM((2, M, BLK), jnp.float32),
            pltpu.VMEM((2, HALF, BLK), jnp.float32),
            pltpu.VMEM((2, HALF, BLK), jnp.float32),
            pltpu.SemaphoreType.DMA((2,)),
            pltpu.SemaphoreType.DMA((2,)),
            pltpu.SemaphoreType.DMA((2,)),
            pltpu.SemaphoreType.DMA((2,)),
            pltpu.SemaphoreType.DMA((2,)),
            pltpu.SemaphoreType.DMA((4,)),
            pltpu.SemaphoreType.DMA((4,)),
            pltpu.SemaphoreType.DMA((2,)),
        ],
        compiler_params=pltpu.CompilerParams(
            collective_id=0,
            vmem_limit_bytes=100 * 1024 * 1024,
        ),
    )(x, dy)


# device time: 246477 ns/iter; 2.0616x vs baseline; 1.0549x over previous
import jax
import jax.numpy as jnp
from jax import lax
from jax.experimental import pallas as pl
from jax.experimental.pallas import tpu as pltpu

M = 2048
K = 2048
N = 8192
HCOLS = N // 2
BLK = 512
NBLK = HCOLS // BLK
HALF = M // 2


def kernel(x, dy):
    def body(x_ref, dy_hbm, out_ref,
             dy_v, partial, zrecv, s_buf,
             dy_sems, z_send_sems, z_recv_sems,
             px_send_sems, py_send_sems, px_recv_sems, py_recv_sems,
             store_sems):
        my_x = lax.axis_index("x")
        my_y = lax.axis_index("y")
        my_z = lax.axis_index("z")
        H = (my_x + my_y) % 2
        col0 = H * HCOLS
        ccol0 = (1 - H) * HCOLS
        z_nbr = (my_x, my_y, 1 - my_z)
        x_nbr = (1 - my_x, my_y, my_z)
        y_nbr = (my_x, 1 - my_y, my_z)

        barrier = pltpu.get_barrier_semaphore()
        for nbr in (z_nbr, x_nbr, y_nbr):
            pl.semaphore_signal(
                barrier, inc=1, device_id=nbr,
                device_id_type=pltpu.DeviceIdType.MESH,
            )
        pl.semaphore_wait(barrier, 3)

        def dy_copy(b):
            return pltpu.make_async_copy(
                dy_hbm.at[:, pl.ds(col0 + b * BLK, BLK)],
                dy_v.at[b % 2],
                dy_sems.at[b % 2],
            )

        def z_rdma(b):
            return pltpu.make_async_remote_copy(
                src_ref=partial.at[b % 2, pl.ds((1 - my_z) * HALF, HALF), :],
                dst_ref=zrecv.at[b % 4],
                send_sem=z_send_sems.at[b % 4],
                recv_sem=z_recv_sems.at[b % 4],
                device_id=z_nbr,
                device_id_type=pltpu.DeviceIdType.MESH,
            )

        def plane_rdma(b):
            nbr = x_nbr if b < NBLK // 2 else y_nbr
            ssems = px_send_sems if b < NBLK // 2 else py_send_sems
            rsems = px_recv_sems if b < NBLK // 2 else py_recv_sems
            return pltpu.make_async_remote_copy(
                src_ref=s_buf.at[b % 2],
                dst_ref=out_ref.at[:, pl.ds(col0 + b * BLK, BLK)],
                send_sem=ssems.at[b % 2],
                recv_sem=rsems.at[b % (NBLK // 2)],
                device_id=nbr,
                device_id_type=pltpu.DeviceIdType.MESH,
            )

        def store_copy(b):
            return pltpu.make_async_copy(
                s_buf.at[b % 2],
                out_ref.at[:, pl.ds(col0 + b * BLK, BLK)],
                store_sems.at[b % 2],
            )

        def finalize(b):
            z_rdma(b).wait_recv()
            if b >= 2:
                plane_rdma(b - 2).wait_send()
                store_copy(b - 2).wait()
            s_buf[b % 2] = (
                partial[b % 2, pl.ds(my_z * HALF, HALF), :] + zrecv[b % 4]
            )
            store_copy(b).start()
            plane_rdma(b).start()

        dy_copy(0).start()
        for b in range(NBLK):
            if b + 1 < NBLK:
                dy_copy(b + 1).start()
            dy_copy(b).wait()
            if b >= 2:
                finalize(b - 2)
                z_rdma(b - 2).wait_send()
            partial[b % 2] = lax.dot_general(
                x_ref[...], dy_v[b % 2],
                (((0,), (0,)), ((), ())),
                preferred_element_type=jnp.float32,
            )
            z_rdma(b).start()
        finalize(NBLK - 2)
        finalize(NBLK - 1)

        z_rdma(NBLK - 2).wait_send()
        z_rdma(NBLK - 1).wait_send()
        for b in (NBLK - 2, NBLK - 1):
            plane_rdma(b).wait_send()
            store_copy(b).wait()

        for b in range(NBLK):
            nbr = x_nbr if b < NBLK // 2 else y_nbr
            rsems = px_recv_sems if b < NBLK // 2 else py_recv_sems
            recv = pltpu.make_async_remote_copy(
                src_ref=s_buf.at[b % 2],
                dst_ref=out_ref.at[:, pl.ds(ccol0 + b * BLK, BLK)],
                send_sem=px_send_sems.at[b % 2],
                recv_sem=rsems.at[b % (NBLK // 2)],
                device_id=nbr,
                device_id_type=pltpu.DeviceIdType.MESH,
            )
            recv.wait_recv()

    return pl.pallas_call(
        body,
        in_specs=[
            pl.BlockSpec(memory_space=pltpu.MemorySpace.VMEM),
            pl.BlockSpec(memory_space=pl.ANY),
        ],
        out_specs=pl.BlockSpec(memory_space=pl.ANY),
        out_shape=jax.ShapeDtypeStruct((HALF, N), jnp.float32),
        scratch_shapes=[
            pltpu.VMEM((2, K, BLK), jnp.float32),
            pltpu.VMEM((2, M, BLK), jnp.float32),
            pltpu.VMEM((2, HALF, BLK), jnp.float32),
            pltpu.VMEM((2, HALF, BLK), jnp.float32),
            pltpu.SemaphoreType.DMA((2,)),
            pltpu.SemaphoreType.DMA((2,)),
            pltpu.SemaphoreType.DMA((2,)),
            pltpu.SemaphoreType.DMA((2,)),
            pltpu.SemaphoreType.DMA((2,)),
            pltpu.SemaphoreType.DMA((4,)),
            pltpu.SemaphoreType.DMA((4,)),
            pltpu.SemaphoreType.DMA((2,)),
        ],
        compiler_params=pltpu.CompilerParams(
            collective_id=0,
            vmem_limit_bytes=100 * 1024 * 1024,
        ),
    )(x, dy)


# device time: 201718 ns/iter; 2.5191x vs baseline; 1.2219x over previous
import jax
import jax.numpy as jnp
from jax import lax
from jax.experimental import pallas as pl
from jax.experimental.pallas import tpu as pltpu

M = 2048
K = 2048
N = 8192
QCOLS = N // 4
BLK = 512
NB = QCOLS // BLK
HALF = M // 2

_MESH = pltpu.DeviceIdType.MESH


def kernel(x, dy):
    def body(x_ref, dy_hbm, out_ref,
             dy_v, partial, zrecv, s_buf, fwd_x, fwd_y,
             dy_sems, z_send_sems, z_recv_sems,
             ps_x, ps_y, fs_x, fs_y,
             xr_out, xr_fwd, xr_diag, yr_out, yr_fwd, yr_diag,
             store_sems, fcopy_sems):
        my_x = lax.axis_index("x")
        my_y = lax.axis_index("y")
        my_z = lax.axis_index("z")
        z_nbr = (my_x, my_y, 1 - my_z)
        x_nbr = (1 - my_x, my_y, my_z)
        y_nbr = (my_x, 1 - my_y, my_z)
        c_me = (2 * my_x + my_y) * QCOLS
        c_x = (2 * (1 - my_x) + my_y) * QCOLS
        c_y = (2 * my_x + (1 - my_y)) * QCOLS
        c_d = (2 * (1 - my_x) + (1 - my_y)) * QCOLS

        barrier = pltpu.get_barrier_semaphore()
        for nbr in (z_nbr, x_nbr, y_nbr):
            pl.semaphore_signal(barrier, inc=1, device_id=nbr,
                                device_id_type=_MESH)
        pl.semaphore_wait(barrier, 3)

        def dy_copy(b):
            return pltpu.make_async_copy(
                dy_hbm.at[:, pl.ds(c_me + b * BLK, BLK)],
                dy_v.at[b % 2],
                dy_sems.at[b % 2],
            )

        def z_rdma(b):
            return pltpu.make_async_remote_copy(
                src_ref=partial.at[b % 2, pl.ds((1 - my_z) * HALF, HALF), :],
                dst_ref=zrecv.at[b],
                send_sem=z_send_sems.at[b],
                recv_sem=z_recv_sems.at[b],
                device_id=z_nbr,
                device_id_type=_MESH,
            )

        def plane_x_rdma(b):
            if b < 2:
                dst, rsem = out_ref.at[:, pl.ds(c_me + b * BLK, BLK)], xr_out.at[b]
            else:
                dst, rsem = fwd_x.at[b - 2], xr_fwd.at[b - 2]
            return pltpu.make_async_remote_copy(
                src_ref=s_buf.at[b], dst_ref=dst,
                send_sem=ps_x.at[b], recv_sem=rsem,
                device_id=x_nbr, device_id_type=_MESH,
            )

        def plane_y_rdma(b):
            if b >= 2:
                dst, rsem = out_ref.at[:, pl.ds(c_me + b * BLK, BLK)], yr_out.at[b - 2]
            else:
                dst, rsem = fwd_y.at[b], yr_fwd.at[b]
            return pltpu.make_async_remote_copy(
                src_ref=s_buf.at[b], dst_ref=dst,
                send_sem=ps_y.at[b], recv_sem=rsem,
                device_id=y_nbr, device_id_type=_MESH,
            )

        def store_copy(b):
            return pltpu.make_async_copy(
                s_buf.at[b],
                out_ref.at[:, pl.ds(c_me + b * BLK, BLK)],
                store_sems.at[b],
            )

        def fwdx_copy(f):
            return pltpu.make_async_copy(
                fwd_x.at[f],
                out_ref.at[:, pl.ds(c_x + (2 + f) * BLK, BLK)],
                fcopy_sems.at[f],
            )

        def fwdy_copy(f):
            return pltpu.make_async_copy(
                fwd_y.at[f],
                out_ref.at[:, pl.ds(c_y + f * BLK, BLK)],
                fcopy_sems.at[2 + f],
            )

        def fwdx_send(f):
            return pltpu.make_async_remote_copy(
                src_ref=fwd_x.at[f],
                dst_ref=out_ref.at[:, pl.ds(c_x + (2 + f) * BLK, BLK)],
                send_sem=fs_y.at[f], recv_sem=yr_diag.at[f],
                device_id=y_nbr, device_id_type=_MESH,
            )

        def fwdy_send(f):
            return pltpu.make_async_remote_copy(
                src_ref=fwd_y.at[f],
                dst_ref=out_ref.at[:, pl.ds(c_y + f * BLK, BLK)],
                send_sem=fs_x.at[f], recv_sem=xr_diag.at[f],
                device_id=x_nbr, device_id_type=_MESH,
            )

        def finalize(b):
            z_rdma(b).wait_recv()
            s_buf[b] = partial[b % 2, pl.ds(my_z * HALF, HALF), :] + zrecv[b]
            store_copy(b).start()
            plane_x_rdma(b).start()
            plane_y_rdma(b).start()

        dy_copy(0).start()
        for b in range(NB):
            if b + 1 < NB:
                dy_copy(b + 1).start()
            dy_copy(b).wait()
            if b >= 2:
                finalize(b - 2)
                z_rdma(b - 2).wait_send()
            partial[b % 2] = lax.dot_general(
                x_ref[...], dy_v[b % 2],
                (((0,), (0,)), ((), ())),
                preferred_element_type=jnp.float32,
            )
            z_rdma(b).start()
        finalize(NB - 2)
        finalize(NB - 1)

        for f in range(2):
            plane_x_rdma(2 + f).wait_recv()
            fwdx_copy(f).start()
            fwdx_send(f).start()
            plane_y_rdma(f).wait_recv()
            fwdy_copy(f).start()
            fwdy_send(f).start()

        for b in (NB - 2, NB - 1):
            z_rdma(b).wait_send()
        for b in range(NB):
            plane_x_rdma(b).wait_send()
            plane_y_rdma(b).wait_send()
            store_copy(b).wait()
        for f in range(2):
            fwdx_send(f).wait_send()
            fwdy_send(f).wait_send()
            fwdx_copy(f).wait()
            fwdy_copy(f).wait()
            pltpu.make_async_remote_copy(
                src_ref=s_buf.at[f],
                dst_ref=out_ref.at[:, pl.ds(c_x + f * BLK, BLK)],
                send_sem=ps_x.at[f],
                recv_sem=xr_out.at[f],
                device_id=x_nbr, device_id_type=_MESH,
            ).wait_recv()
            pltpu.make_async_remote_copy(
                src_ref=s_buf.at[f],
                dst_ref=out_ref.at[:, pl.ds(c_y + (2 + f) * BLK, BLK)],
                send_sem=ps_y.at[f],
                recv_sem=yr_out.at[f],
                device_id=y_nbr, device_id_type=_MESH,
            ).wait_recv()
            pltpu.make_async_remote_copy(
                src_ref=s_buf.at[f],
                dst_ref=out_ref.at[:, pl.ds(c_d + f * BLK, BLK)],
                send_sem=ps_x.at[f],
                recv_sem=xr_diag.at[f],
                device_id=x_nbr, device_id_type=_MESH,
            ).wait_recv()
            pltpu.make_async_remote_copy(
                src_ref=s_buf.at[f],
                dst_ref=out_ref.at[:, pl.ds(c_d + (2 + f) * BLK, BLK)],
                send_sem=ps_y.at[f],
                recv_sem=yr_diag.at[f],
                device_id=y_nbr, device_id_type=_MESH,
            ).wait_recv()

    return pl.pallas_call(
        body,
        in_specs=[
            pl.BlockSpec(memory_space=pltpu.MemorySpace.VMEM),
            pl.BlockSpec(memory_space=pl.ANY),
        ],
        out_specs=pl.BlockSpec(memory_space=pl.ANY),
        out_shape=jax.ShapeDtypeStruct((HALF, N), jnp.float32),
        scratch_shapes=[
            pltpu.VMEM((2, K, BLK), jnp.float32),
            pltpu.VMEM((2, M, BLK), jnp.float32),
            pltpu.VMEM((NB, HALF, BLK), jnp.float32),
            pltpu.VMEM((NB, HALF, BLK), jnp.float32),
            pltpu.VMEM((2, HALF, BLK), jnp.float32),
            pltpu.VMEM((2, HALF, BLK), jnp.float32),
            pltpu.SemaphoreType.DMA((2,)),
            pltpu.SemaphoreType.DMA((NB,)),
            pltpu.SemaphoreType.DMA((NB,)),
            pltpu.SemaphoreType.DMA((NB,)),
            pltpu.SemaphoreType.DMA((NB,)),
            pltpu.SemaphoreType.DMA((2,)),
            pltpu.SemaphoreType.DMA((2,)),
            pltpu.SemaphoreType.DMA((2,)),
            pltpu.SemaphoreType.DMA((2,)),
            pltpu.SemaphoreType.DMA((2,)),
            pltpu.SemaphoreType.DMA((2,)),
            pltpu.SemaphoreType.DMA((2,)),
            pltpu.SemaphoreType.DMA((2,)),
            pltpu.SemaphoreType.DMA((NB,)),
            pltpu.SemaphoreType.DMA((4,)),
        ],
        compiler_params=pltpu.CompilerParams(
            collective_id=0,
            vmem_limit_bytes=100 * 1024 * 1024,
        ),
    )(x, dy)


# device time: 201494 ns/iter; 2.5219x vs baseline; 1.0011x over previous
import jax
import jax.numpy as jnp
from jax import lax
from jax.experimental import pallas as pl
from jax.experimental.pallas import tpu as pltpu

M = 2048
K = 2048
N = 8192
QCOLS = N // 4
BLK = 512
NB = QCOLS // BLK
SUB = 4
SBLK = BLK // SUB
HALF = M // 2

_MESH = pltpu.DeviceIdType.MESH


def kernel(x, dy):
    def body(x_ref, dy_hbm, out_ref,
             dy_v, partial, zrecv, s_buf, fwd_x, fwd_y,
             dy_sems, z_send_sems, z_recv_sems, z0_send_sems, z0_recv_sems,
             ps_x, ps_y, fs_x, fs_y,
             xr_out, xr_fwd, xr_diag, yr_out, yr_fwd, yr_diag,
             store_sems, fcopy_sems):
        my_x = lax.axis_index("x")
        my_y = lax.axis_index("y")
        my_z = lax.axis_index("z")
        z_nbr = (my_x, my_y, 1 - my_z)
        x_nbr = (1 - my_x, my_y, my_z)
        y_nbr = (my_x, 1 - my_y, my_z)
        c_me = (2 * my_x + my_y) * QCOLS
        c_x = (2 * (1 - my_x) + my_y) * QCOLS
        c_y = (2 * my_x + (1 - my_y)) * QCOLS
        c_d = (2 * (1 - my_x) + (1 - my_y)) * QCOLS

        barrier = pltpu.get_barrier_semaphore()
        for nbr in (z_nbr, x_nbr, y_nbr):
            pl.semaphore_signal(barrier, inc=1, device_id=nbr,
                                device_id_type=_MESH)
        pl.semaphore_wait(barrier, 3)

        def dy_copy(b):
            return pltpu.make_async_copy(
                dy_hbm.at[:, pl.ds(c_me + b * BLK, BLK)],
                dy_v.at[b % 2],
                dy_sems.at[b % 2],
            )

        def z_rdma(b):
            return pltpu.make_async_remote_copy(
                src_ref=partial.at[b % 2, pl.ds((1 - my_z) * HALF, HALF), :],
                dst_ref=zrecv.at[b],
                send_sem=z_send_sems.at[b],
                recv_sem=z_recv_sems.at[b],
                device_id=z_nbr,
                device_id_type=_MESH,
            )

        def z0_sub(k):
            return pltpu.make_async_remote_copy(
                src_ref=partial.at[0, pl.ds((1 - my_z) * HALF, HALF),
                                   pl.ds(k * SBLK, SBLK)],
                dst_ref=zrecv.at[0, :, pl.ds(k * SBLK, SBLK)],
                send_sem=z0_send_sems.at[k],
                recv_sem=z0_recv_sems.at[k],
                device_id=z_nbr,
                device_id_type=_MESH,
            )

        def plane_x_rdma(b):
            if b < 2:
                dst, rsem = out_ref.at[:, pl.ds(c_me + b * BLK, BLK)], xr_out.at[b]
            else:
                dst, rsem = fwd_x.at[b - 2], xr_fwd.at[b - 2]
            return pltpu.make_async_remote_copy(
                src_ref=s_buf.at[b], dst_ref=dst,
                send_sem=ps_x.at[b], recv_sem=rsem,
                device_id=x_nbr, device_id_type=_MESH,
            )

        def plane_y_rdma(b):
            if b >= 2:
                dst, rsem = out_ref.at[:, pl.ds(c_me + b * BLK, BLK)], yr_out.at[b - 2]
            else:
                dst, rsem = fwd_y.at[b], yr_fwd.at[b]
            return pltpu.make_async_remote_copy(
                src_ref=s_buf.at[b], dst_ref=dst,
                send_sem=ps_y.at[b], recv_sem=rsem,
                device_id=y_nbr, device_id_type=_MESH,
            )

        def store_copy(b):
            return pltpu.make_async_copy(
                s_buf.at[b],
                out_ref.at[:, pl.ds(c_me + b * BLK, BLK)],
                store_sems.at[b],
            )

        def fwdx_copy(f):
            return pltpu.make_async_copy(
                fwd_x.at[f],
                out_ref.at[:, pl.ds(c_x + (2 + f) * BLK, BLK)],
                fcopy_sems.at[f],
            )

        def fwdy_copy(f):
            return pltpu.make_async_copy(
                fwd_y.at[f],
                out_ref.at[:, pl.ds(c_y + f * BLK, BLK)],
                fcopy_sems.at[2 + f],
            )

        def fwdx_send(f):
            return pltpu.make_async_remote_copy(
                src_ref=fwd_x.at[f],
                dst_ref=out_ref.at[:, pl.ds(c_x + (2 + f) * BLK, BLK)],
                send_sem=fs_y.at[f], recv_sem=yr_diag.at[f],
                device_id=y_nbr, device_id_type=_MESH,
            )

        def fwdy_send(f):
            return pltpu.make_async_remote_copy(
                src_ref=fwd_y.at[f],
                dst_ref=out_ref.at[:, pl.ds(c_y + f * BLK, BLK)],
                send_sem=fs_x.at[f], recv_sem=xr_diag.at[f],
                device_id=x_nbr, device_id_type=_MESH,
            )

        def finalize(b):
            if b == 0:
                for k in range(SUB):
                    z0_sub(k).wait_recv()
            else:
                z_rdma(b).wait_recv()
            s_buf[b] = partial[b % 2, pl.ds(my_z * HALF, HALF), :] + zrecv[b]
            store_copy(b).start()
            plane_x_rdma(b).start()
            plane_y_rdma(b).start()

        dy_copy(0).start()
        dy_copy(1).start()
        dy_copy(0).wait()
        for k in range(SUB):
            partial[0, :, k * SBLK:(k + 1) * SBLK] = lax.dot_general(
                x_ref[...], dy_v[0, :, k * SBLK:(k + 1) * SBLK],
                (((0,), (0,)), ((), ())),
                preferred_element_type=jnp.float32,
            )
            z0_sub(k).start()
        for b in range(1, NB):
            if b + 1 < NB:
                dy_copy(b + 1).start()
            dy_copy(b).wait()
            if b >= 2:
                finalize(b - 2)
                if b == 2:
                    for k in range(SUB):
                        z0_sub(k).wait_send()
                else:
                    z_rdma(b - 2).wait_send()
            partial[b % 2] = lax.dot_general(
                x_ref[...], dy_v[b % 2],
                (((0,), (0,)), ((), ())),
                preferred_element_type=jnp.float32,
            )
            z_rdma(b).start()
        finalize(NB - 2)
        finalize(NB - 1)

        for f in range(2):
            plane_x_rdma(2 + f).wait_recv()
            fwdx_copy(f).start()
            fwdx_send(f).start()
            plane_y_rdma(f).wait_recv()
            fwdy_copy(f).start()
            fwdy_send(f).start()

        for b in (NB - 2, NB - 1):
            z_rdma(b).wait_send()
        for b in range(NB):
            plane_x_rdma(b).wait_send()
            plane_y_rdma(b).wait_send()
            store_copy(b).wait()
        for f in range(2):
            fwdx_send(f).wait_send()
            fwdy_send(f).wait_send()
            fwdx_copy(f).wait()
            fwdy_copy(f).wait()
            pltpu.make_async_remote_copy(
                src_ref=s_buf.at[f],
                dst_ref=out_ref.at[:, pl.ds(c_x + f * BLK, BLK)],
                send_sem=ps_x.at[f],
                recv_sem=xr_out.at[f],
                device_id=x_nbr, device_id_type=_MESH,
            ).wait_recv()
            pltpu.make_async_remote_copy(
                src_ref=s_buf.at[f],
                dst_ref=out_ref.at[:, pl.ds(c_y + (2 + f) * BLK, BLK)],
                send_sem=ps_y.at[f],
                recv_sem=yr_out.at[f],
                device_id=y_nbr, device_id_type=_MESH,
            ).wait_recv()
            pltpu.make_async_remote_copy(
                src_ref=s_buf.at[f],
                dst_ref=out_ref.at[:, pl.ds(c_d + f * BLK, BLK)],
                send_sem=ps_x.at[f],
                recv_sem=xr_diag.at[f],
                device_id=x_nbr, device_id_type=_MESH,
            ).wait_recv()
            pltpu.make_async_remote_copy(
                src_ref=s_buf.at[f],
                dst_ref=out_ref.at[:, pl.ds(c_d + (2 + f) * BLK, BLK)],
                send_sem=ps_y.at[f],
                recv_sem=yr_diag.at[f],
                device_id=y_nbr, device_id_type=_MESH,
            ).wait_recv()

    return pl.pallas_call(
        body,
        in_specs=[
            pl.BlockSpec(memory_space=pltpu.MemorySpace.VMEM),
            pl.BlockSpec(memory_space=pl.ANY),
        ],
        out_specs=pl.BlockSpec(memory_space=pl.ANY),
        out_shape=jax.ShapeDtypeStruct((HALF, N), jnp.float32),
        scratch_shapes=[
            pltpu.VMEM((2, K, BLK), jnp.float32),
            pltpu.VMEM((2, M, BLK), jnp.float32),
            pltpu.VMEM((NB, HALF, BLK), jnp.float32),
            pltpu.VMEM((NB, HALF, BLK), jnp.float32),
            pltpu.VMEM((2, HALF, BLK), jnp.float32),
            pltpu.VMEM((2, HALF, BLK), jnp.float32),
            pltpu.SemaphoreType.DMA((2,)),
            pltpu.SemaphoreType.DMA((NB,)),
            pltpu.SemaphoreType.DMA((NB,)),
            pltpu.SemaphoreType.DMA((SUB,)),
            pltpu.SemaphoreType.DMA((SUB,)),
            pltpu.SemaphoreType.DMA((NB,)),
            pltpu.SemaphoreType.DMA((NB,)),
            pltpu.SemaphoreType.DMA((2,)),
            pltpu.SemaphoreType.DMA((2,)),
            pltpu.SemaphoreType.DMA((2,)),
            pltpu.SemaphoreType.DMA((2,)),
            pltpu.SemaphoreType.DMA((2,)),
            pltpu.SemaphoreType.DMA((2,)),
            pltpu.SemaphoreType.DMA((2,)),
            pltpu.SemaphoreType.DMA((2,)),
            pltpu.SemaphoreType.DMA((NB,)),
            pltpu.SemaphoreType.DMA((4,)),
        ],
        compiler_params=pltpu.CompilerParams(
            collective_id=0,
            vmem_limit_bytes=100 * 1024 * 1024,
        ),
    )(x, dy)


# device time: 199496 ns/iter; 2.5472x vs baseline; 1.0100x over previous
import jax
import jax.numpy as jnp
from jax import lax
from jax.experimental import pallas as pl
from jax.experimental.pallas import tpu as pltpu

M = 2048
K = 2048
N = 8192
QCOLS = N // 4
BLK = 512
NB = QCOLS // BLK
SUB = 4
SBLK = BLK // SUB
HALF = M // 2

_MESH = pltpu.DeviceIdType.MESH


def kernel(x, dy):
    def body(x_hbm, dy_hbm, out_ref,
             x_ref, dy_v, partial, zrecv, s_buf, fwd_x, fwd_y,
             dy_sems, z_send_sems, z_recv_sems, z0_send_sems, z0_recv_sems,
             ps_x, ps_y, fs_x, fs_y,
             xr_out, xr_fwd, xr_diag, yr_out, yr_fwd, yr_diag,
             store_sems, fcopy_sems, x_sem):
        my_x = lax.axis_index("x")
        my_y = lax.axis_index("y")
        my_z = lax.axis_index("z")
        z_nbr = (my_x, my_y, 1 - my_z)
        x_nbr = (1 - my_x, my_y, my_z)
        y_nbr = (my_x, 1 - my_y, my_z)
        c_me = (2 * my_x + my_y) * QCOLS
        c_x = (2 * (1 - my_x) + my_y) * QCOLS
        c_y = (2 * my_x + (1 - my_y)) * QCOLS
        c_d = (2 * (1 - my_x) + (1 - my_y)) * QCOLS

        x_copy = pltpu.make_async_copy(x_hbm, x_ref, x_sem)
        x_copy.start()

        barrier = pltpu.get_barrier_semaphore()
        for nbr in (z_nbr, x_nbr, y_nbr):
            pl.semaphore_signal(barrier, inc=1, device_id=nbr,
                                device_id_type=_MESH)
        pl.semaphore_wait(barrier, 3)

        def dy_copy(b):
            return pltpu.make_async_copy(
                dy_hbm.at[:, pl.ds(c_me + b * BLK, BLK)],
                dy_v.at[b % 2],
                dy_sems.at[b % 2],
            )

        def z_rdma(b):
            return pltpu.make_async_remote_copy(
                src_ref=partial.at[b % 2, pl.ds((1 - my_z) * HALF, HALF), :],
                dst_ref=zrecv.at[b],
                send_sem=z_send_sems.at[b],
                recv_sem=z_recv_sems.at[b],
                device_id=z_nbr,
                device_id_type=_MESH,
            )

        def z0_sub(k):
            return pltpu.make_async_remote_copy(
                src_ref=partial.at[0, pl.ds((1 - my_z) * HALF, HALF),
                                   pl.ds(k * SBLK, SBLK)],
                dst_ref=zrecv.at[0, :, pl.ds(k * SBLK, SBLK)],
                send_sem=z0_send_sems.at[k],
                recv_sem=z0_recv_sems.at[k],
                device_id=z_nbr,
                device_id_type=_MESH,
            )

        def plane_x_rdma(b):
            if b < 2:
                dst, rsem = out_ref.at[:, pl.ds(c_me + b * BLK, BLK)], xr_out.at[b]
            else:
                dst, rsem = fwd_x.at[b - 2], xr_fwd.at[b - 2]
            return pltpu.make_async_remote_copy(
                src_ref=s_buf.at[b], dst_ref=dst,
                send_sem=ps_x.at[b], recv_sem=rsem,
                device_id=x_nbr, device_id_type=_MESH,
            )

        def plane_y_rdma(b):
            if b >= 2:
                dst, rsem = out_ref.at[:, pl.ds(c_me + b * BLK, BLK)], yr_out.at[b - 2]
            else:
                dst, rsem = fwd_y.at[b], yr_fwd.at[b]
            return pltpu.make_async_remote_copy(
                src_ref=s_buf.at[b], dst_ref=dst,
                send_sem=ps_y.at[b], recv_sem=rsem,
                device_id=y_nbr, device_id_type=_MESH,
            )

        def store_copy(b):
            return pltpu.make_async_copy(
                s_buf.at[b],
                out_ref.at[:, pl.ds(c_me + b * BLK, BLK)],
                store_sems.at[b],
            )

        def fwdx_copy(f):
            return pltpu.make_async_copy(
                fwd_x.at[f],
                out_ref.at[:, pl.ds(c_x + (2 + f) * BLK, BLK)],
                fcopy_sems.at[f],
            )

        def fwdy_copy(f):
            return pltpu.make_async_copy(
                fwd_y.at[f],
                out_ref.at[:, pl.ds(c_y + f * BLK, BLK)],
                fcopy_sems.at[2 + f],
            )

        def fwdx_send(f):
            return pltpu.make_async_remote_copy(
                src_ref=fwd_x.at[f],
                dst_ref=out_ref.at[:, pl.ds(c_x + (2 + f) * BLK, BLK)],
                send_sem=fs_y.at[f], recv_sem=yr_diag.at[f],
                device_id=y_nbr, device_id_type=_MESH,
            )

        def fwdy_send(f):
            return pltpu.make_async_remote_copy(
                src_ref=fwd_y.at[f],
                dst_ref=out_ref.at[:, pl.ds(c_y + f * BLK, BLK)],
                send_sem=fs_x.at[f], recv_sem=xr_diag.at[f],
                device_id=x_nbr, device_id_type=_MESH,
            )

        def finalize(b):
            if b == 0:
                for k in range(SUB):
                    z0_sub(k).wait_recv()
            else:
                z_rdma(b).wait_recv()
            s_buf[b] = partial[b % 2, pl.ds(my_z * HALF, HALF), :] + zrecv[b]
            store_copy(b).start()
            plane_x_rdma(b).start()
            plane_y_rdma(b).start()

        dy_copy(0).start()
        dy_copy(1).start()
        dy_copy(0).wait()
        x_copy.wait()
        for k in range(SUB):
            partial[0, :, k * SBLK:(k + 1) * SBLK] = lax.dot_general(
                x_ref[...], dy_v[0, :, k * SBLK:(k + 1) * SBLK],
                (((0,), (0,)), ((), ())),
                preferred_element_type=jnp.float32,
            )
            z0_sub(k).start()
        for b in range(1, NB):
            if b + 1 < NB:
                dy_copy(b + 1).start()
            dy_copy(b).wait()
            if b >= 2:
                finalize(b - 2)
                if b == 2:
                    for k in range(SUB):
                        z0_sub(k).wait_send()
                else:
                    z_rdma(b - 2).wait_send()
            partial[b % 2] = lax.dot_general(
                x_ref[...], dy_v[b % 2],
                (((0,), (0,)), ((), ())),
                preferred_element_type=jnp.float32,
            )
            z_rdma(b).start()
        finalize(NB - 2)
        finalize(NB - 1)

        for f in range(2):
            plane_x_rdma(2 + f).wait_recv()
            fwdx_copy(f).start()
            fwdx_send(f).start()
            plane_y_rdma(f).wait_recv()
            fwdy_copy(f).start()
            fwdy_send(f).start()

        for b in (NB - 2, NB - 1):
            z_rdma(b).wait_send()
        for b in range(NB):
            plane_x_rdma(b).wait_send()
            plane_y_rdma(b).wait_send()
            store_copy(b).wait()
        for f in range(2):
            fwdx_send(f).wait_send()
            fwdy_send(f).wait_send()
            fwdx_copy(f).wait()
            fwdy_copy(f).wait()
            pltpu.make_async_remote_copy(
                src_ref=s_buf.at[f],
                dst_ref=out_ref.at[:, pl.ds(c_x + f * BLK, BLK)],
                send_sem=ps_x.at[f],
                recv_sem=xr_out.at[f],
                device_id=x_nbr, device_id_type=_MESH,
            ).wait_recv()
            pltpu.make_async_remote_copy(
                src_ref=s_buf.at[f],
                dst_ref=out_ref.at[:, pl.ds(c_y + (2 + f) * BLK, BLK)],
                send_sem=ps_y.at[f],
                recv_sem=yr_out.at[f],
                device_id=y_nbr, device_id_type=_MESH,
            ).wait_recv()
            pltpu.make_async_remote_copy(
                src_ref=s_buf.at[f],
                dst_ref=out_ref.at[:, pl.ds(c_d + f * BLK, BLK)],
                send_sem=ps_x.at[f],
                recv_sem=xr_diag.at[f],
                device_id=x_nbr, device_id_type=_MESH,
            ).wait_recv()
            pltpu.make_async_remote_copy(
                src_ref=s_buf.at[f],
                dst_ref=out_ref.at[:, pl.ds(c_d + (2 + f) * BLK, BLK)],
                send_sem=ps_y.at[f],
                recv_sem=yr_diag.at[f],
                device_id=y_nbr, device_id_type=_MESH,
            ).wait_recv()

    return pl.pallas_call(
        body,
        in_specs=[
            pl.BlockSpec(memory_space=pl.ANY),
            pl.BlockSpec(memory_space=pl.ANY),
        ],
        out_specs=pl.BlockSpec(memory_space=pl.ANY),
        out_shape=jax.ShapeDtypeStruct((HALF, N), jnp.float32),
        scratch_shapes=[
            pltpu.VMEM((K, M), jnp.float32),
            pltpu.VMEM((2, K, BLK), jnp.float32),
            pltpu.VMEM((2, M, BLK), jnp.float32),
            pltpu.VMEM((NB, HALF, BLK), jnp.float32),
            pltpu.VMEM((NB, HALF, BLK), jnp.float32),
            pltpu.VMEM((2, HALF, BLK), jnp.float32),
            pltpu.VMEM((2, HALF, BLK), jnp.float32),
            pltpu.SemaphoreType.DMA((2,)),
            pltpu.SemaphoreType.DMA((NB,)),
            pltpu.SemaphoreType.DMA((NB,)),
            pltpu.SemaphoreType.DMA((SUB,)),
            pltpu.SemaphoreType.DMA((SUB,)),
            pltpu.SemaphoreType.DMA((NB,)),
            pltpu.SemaphoreType.DMA((NB,)),
            pltpu.SemaphoreType.DMA((2,)),
            pltpu.SemaphoreType.DMA((2,)),
            pltpu.SemaphoreType.DMA((2,)),
            pltpu.SemaphoreType.DMA((2,)),
            pltpu.SemaphoreType.DMA((2,)),
            pltpu.SemaphoreType.DMA((2,)),
            pltpu.SemaphoreType.DMA((2,)),
            pltpu.SemaphoreType.DMA((2,)),
            pltpu.SemaphoreType.DMA((NB,)),
            pltpu.SemaphoreType.DMA((4,)),
            pltpu.SemaphoreType.DMA,
        ],
        compiler_params=pltpu.CompilerParams(
            collective_id=0,
            vmem_limit_bytes=100 * 1024 * 1024,
        ),
    )(x, dy)
